# baseline (speedup 1.0000x reference)
import functools

import jax
import jax.numpy as jnp
import numpy as np
from jax import lax
from jax.experimental import pallas as pl
from jax.experimental.pallas import tpu as pltpu
from jax.experimental.pallas import tpu_sc as plsc

N = 4096
D = 128
E = 65536
K = 2048
NEG_SLOPE = 0.2

EP = E + N
NW = 32
EPW = EP // NW
CH = 128
NCH = EPW // CH


def _sc_mesh():
    return plsc.VectorSubcoreMesh(core_axis_name="c", subcore_axis_name="s")


def _wid():
    return lax.axis_index("s") * 2 + lax.axis_index("c")


def _concat_gather(G, i2):
    @functools.partial(
        pl.kernel,
        out_type=jax.ShapeDtypeStruct((2 * EP, D), jnp.float32),
        mesh=_sc_mesh(),
        scratch_types=[
            pltpu.VMEM((2 * CH,), jnp.int32),
            pltpu.VMEM((2 * CH, D), jnp.float32),
            pltpu.SemaphoreType.DMA,
        ],
    )
    def k(G_hbm, i2_hbm, out_hbm, i2b, gb, sem):
        base_w = _wid() * EPW

        def body(ch, c):
            eb = base_w + ch * CH
            pltpu.sync_copy(i2_hbm.at[pl.ds(2 * eb, 2 * CH)], i2b)
            pltpu.async_copy(G_hbm.at[i2b], gb, sem).wait()
            pltpu.sync_copy(gb, out_hbm.at[pl.ds(2 * eb, 2 * CH)])
            return c

        lax.fori_loop(0, NCH, body, 0)

    return k(G, i2)


def _edge_counts(row0, col0):
    EW2 = E // NW
    NC2 = EW2 // CH

    @functools.partial(
        pl.kernel,
        out_type=jax.ShapeDtypeStruct((2, 2, N), jnp.float32),
        mesh=_sc_mesh(),
        scratch_types=[
            pltpu.VMEM((CH,), jnp.int32),
            pltpu.VMEM((CH,), jnp.int32),
            pltpu.VMEM((CH,), jnp.float32),
            pltpu.VMEM((CH,), jnp.float32),
            pltpu.VMEM_SHARED((N,), jnp.float32),
            pltpu.VMEM_SHARED((N,), jnp.float32),
            pltpu.VMEM((N // 16,), jnp.float32),
            pltpu.SemaphoreType.DMA,
            pltpu.SemaphoreType.DMA,
        ],
    )
    def k(row_hbm, col_hbm, out_hbm, rowb, colb, onesb, loopb, sh_r, sh_l, zb, sem, sem2):
        cid = lax.axis_index("c")
        sid = lax.axis_index("s")
        wid = sid * 2 + cid
        for gg in range(N // 16 // 16):
            zb[pl.ds(gg * 16, 16)] = jnp.zeros((16,), jnp.float32)
        for gg in range(CH // 16):
            onesb[pl.ds(gg * 16, 16)] = jnp.ones((16,), jnp.float32)
        pltpu.sync_copy(zb, sh_r.at[pl.ds(sid * (N // 16), N // 16)])
        pltpu.sync_copy(zb, sh_l.at[pl.ds(sid * (N // 16), N // 16)])
        plsc.subcore_barrier()

        def body(ch, c):
            eb = wid * EW2 + ch * CH
            pltpu.sync_copy(row_hbm.at[pl.ds(eb, CH)], rowb)
            pltpu.sync_copy(col_hbm.at[pl.ds(eb, CH)], colb)
            for g in range(CH // 16):
                sl = pl.ds(g * 16, 16)
                loopb[sl] = jnp.where(rowb[sl] == colb[sl], 1.0, 0.0)
            pltpu.async_copy(onesb, sh_r.at[rowb], sem, add=True)
            pltpu.async_copy(loopb, sh_l.at[rowb], sem2, add=True)
            pltpu.make_async_copy(onesb, sh_r.at[rowb], sem).wait()
            pltpu.make_async_copy(loopb, sh_l.at[rowb], sem2).wait()
            return c

        lax.fori_loop(0, NC2, body, 0)
        plsc.subcore_barrier()

        @pl.when(sid == 0)
        def _():
            pltpu.sync_copy(sh_r, out_hbm.at[cid, 0])
            pltpu.sync_copy(sh_l, out_hbm.at[cid, 1])

    return k(row0, col0)


def _gather_scalar(src, idx):
    @functools.partial(
        pl.kernel,
        out_type=jax.ShapeDtypeStruct((EP,), src.dtype),
        mesh=_sc_mesh(),
        scratch_types=[
            pltpu.VMEM((CH,), jnp.int32),
            pltpu.VMEM((CH,), src.dtype),
            pltpu.SemaphoreType.DMA,
        ],
    )
    def k(src_hbm, idx_hbm, out_hbm, idxb, gb, sem):
        base_w = _wid() * EPW

        def body(ch, c):
            eb = base_w + ch * CH
            pltpu.sync_copy(idx_hbm.at[pl.ds(eb, CH)], idxb)
            pltpu.async_copy(src_hbm.at[idxb], gb, sem).wait()
            pltpu.sync_copy(gb, out_hbm.at[pl.ds(eb, CH)])
            return c

        lax.fori_loop(0, NCH, body, 0)

    return k(src, idx)


def _gather_rows(src, idx):
    @functools.partial(
        pl.kernel,
        out_type=jax.ShapeDtypeStruct((EP, D), jnp.float32),
        mesh=_sc_mesh(),
        scratch_types=[
            pltpu.VMEM((CH,), jnp.int32),
            pltpu.VMEM((CH, D), jnp.float32),
            pltpu.SemaphoreType.DMA,
        ],
    )
    def k(src_hbm, idx_hbm, out_hbm, idxb, gb, sem):
        base_w = _wid() * EPW

        def body(ch, c):
            eb = base_w + ch * CH
            pltpu.sync_copy(idx_hbm.at[pl.ds(eb, CH)], idxb)
            pltpu.async_copy(src_hbm.at[idxb], gb, sem).wait()
            pltpu.sync_copy(gb, out_hbm.at[pl.ds(eb, CH)])
            return c

        lax.fori_loop(0, NCH, body, 0)

    return k(src, idx)


def _gather_norm_scale(h, row, col, ew, dinv):
    @functools.partial(
        pl.kernel,
        out_type=jax.ShapeDtypeStruct((EP, D), jnp.float32),
        mesh=_sc_mesh(),
        scratch_types=[
            pltpu.VMEM((CH,), jnp.int32),
            pltpu.VMEM((CH,), jnp.int32),
            pltpu.VMEM((CH,), jnp.float32),
            pltpu.VMEM((CH,), jnp.float32),
            pltpu.VMEM((CH,), jnp.float32),
            pltpu.VMEM((CH, D), jnp.float32),
            pltpu.SemaphoreType.DMA,
            pltpu.SemaphoreType.DMA,
            pltpu.SemaphoreType.DMA,
        ],
    )
    def k(h_hbm, row_hbm, col_hbm, ew_hbm, dinv_hbm, out_hbm,
          rowb, colb, ewb, drb, dcb, gb, sem, sem2, sem3):
        base_w = _wid() * EPW

        def body(ch, c):
            eb = base_w + ch * CH
            pltpu.sync_copy(row_hbm.at[pl.ds(eb, CH)], rowb)
            pltpu.sync_copy(col_hbm.at[pl.ds(eb, CH)], colb)
            pltpu.sync_copy(ew_hbm.at[pl.ds(eb, CH)], ewb)
            pltpu.async_copy(dinv_hbm.at[rowb], drb, sem2)
            pltpu.async_copy(dinv_hbm.at[colb], dcb, sem3)
            pltpu.async_copy(h_hbm.at[colb], gb, sem)
            pltpu.make_async_copy(dinv_hbm.at[rowb], drb, sem2).wait()
            pltpu.make_async_copy(dinv_hbm.at[colb], dcb, sem3).wait()
            for g in range(CH // 16):
                sl = pl.ds(g * 16, 16)
                ewb[sl] = (drb[sl] * ewb[sl]) * dcb[sl]
            pltpu.make_async_copy(h_hbm.at[colb], gb, sem).wait()
            for g in range(CH // 16):
                s16 = ewb[pl.ds(g * 16, 16)]
                for jj in range(16):
                    j = g * 16 + jj
                    sv = jnp.broadcast_to(s16[jj], (16,))
                    for dd in range(D // 16):
                        gb[j, pl.ds(dd * 16, 16)] = gb[j, pl.ds(dd * 16, 16)] * sv
            pltpu.sync_copy(gb, out_hbm.at[pl.ds(eb, CH)])
            return c

        lax.fori_loop(0, NCH, body, 0)

    return k(h, row, col, ew, dinv)


def _gather_scale(src, idx, scale):
    @functools.partial(
        pl.kernel,
        out_type=jax.ShapeDtypeStruct((EP, D), jnp.float32),
        mesh=_sc_mesh(),
        scratch_types=[
            pltpu.VMEM((CH,), jnp.int32),
            pltpu.VMEM((CH,), jnp.float32),
            pltpu.VMEM((CH, D), jnp.float32),
            pltpu.SemaphoreType.DMA,
        ],
    )
    def k(src_hbm, idx_hbm, sc_hbm, out_hbm, idxb, scb, gb, sem):
        base_w = _wid() * EPW

        def body(ch, c):
            eb = base_w + ch * CH
            pltpu.sync_copy(idx_hbm.at[pl.ds(eb, CH)], idxb)
            pltpu.sync_copy(sc_hbm.at[pl.ds(eb, CH)], scb)
            pltpu.async_copy(src_hbm.at[idxb], gb, sem).wait()
            for g in range(CH // 16):
                s16 = scb[pl.ds(g * 16, 16)]
                for jj in range(16):
                    j = g * 16 + jj
                    sv = jnp.broadcast_to(s16[jj], (16,))
                    for dd in range(D // 16):
                        gb[j, pl.ds(dd * 16, 16)] = gb[j, pl.ds(dd * 16, 16)] * sv
            pltpu.sync_copy(gb, out_hbm.at[pl.ds(eb, CH)])
            return c

        lax.fori_loop(0, NCH, body, 0)

    return k(src, idx, scale)


def _matmul_kernel(a_ref, b_ref, o_ref, acc_ref, *, n_k):
    k = pl.program_id(2)

    @pl.when(k == 0)
    def _():
        acc_ref[...] = jnp.zeros_like(acc_ref)

    acc_ref[...] += jnp.dot(a_ref[...], b_ref[...],
                            preferred_element_type=jnp.float32)

    @pl.when(k == n_k - 1)
    def _():
        o_ref[...] = acc_ref[...]


def _matmul(a, b, bm=512, bn=512, bk=512):
    m, k = a.shape
    k2, n = b.shape
    assert k == k2
    grid = (m // bm, n // bn, k // bk)
    return pl.pallas_call(
        functools.partial(_matmul_kernel, n_k=grid[2]),
        grid=grid,
        in_specs=[
            pl.BlockSpec((bm, bk), lambda i, j, kk: (i, kk)),
            pl.BlockSpec((bk, bn), lambda i, j, kk: (kk, j)),
        ],
        out_specs=pl.BlockSpec((bm, bn), lambda i, j, kk: (i, j)),
        out_shape=jax.ShapeDtypeStruct((m, n), jnp.float32),
        scratch_shapes=[pltpu.VMEM((bm, bn), jnp.float32)],
        compiler_params=pltpu.CompilerParams(
            dimension_semantics=("parallel", "parallel", "arbitrary")),
    )(a, b)


def _atb_diag_kernel(a_ref, b_ref, o_ref, acc_ref, *, n_k, bm, bn):
    i = pl.program_id(0)
    j = pl.program_id(1)
    k = pl.program_id(2)

    @pl.when(k == 0)
    def _():
        acc_ref[...] = jnp.zeros_like(acc_ref)

    acc_ref[...] += lax.dot_general(
        a_ref[...], b_ref[...], (((0,), (0,)), ((), ())),
        preferred_element_type=jnp.float32)

    @pl.when(k == n_k - 1)
    def _():
        rows = i * bm + lax.broadcasted_iota(jnp.int32, (bm, bn), 0)
        cols = j * bn + lax.broadcasted_iota(jnp.int32, (bm, bn), 1)
        o_ref[...] = jnp.where(rows == cols, 1.0, acc_ref[...])


def _atb_with_unit_diag(a, b, bm=512, bn=512, bk=512):
    k, m = a.shape
    k2, n = b.shape
    assert k == k2
    grid = (m // bm, n // bn, k // bk)
    return pl.pallas_call(
        functools.partial(_atb_diag_kernel, n_k=grid[2], bm=bm, bn=bn),
        grid=grid,
        in_specs=[
            pl.BlockSpec((bk, bm), lambda i, j, kk: (kk, i)),
            pl.BlockSpec((bk, bn), lambda i, j, kk: (kk, j)),
        ],
        out_specs=pl.BlockSpec((bm, bn), lambda i, j, kk: (i, j)),
        out_shape=jax.ShapeDtypeStruct((m, n), jnp.float32),
        scratch_shapes=[pltpu.VMEM((bm, bn), jnp.float32)],
        compiler_params=pltpu.CompilerParams(
            dimension_semantics=("parallel", "parallel", "arbitrary")),
    )(a, b)


def kernel(x, edge_index, W_gcn, b_gcn, W_q, b_q, W_att, b_att,
           le_w, le1_W, le1_b, le2_W, le2_b):
    row0, col0 = edge_index[0], edge_index[1]
    parts = _edge_counts(row0, col0)
    cnt_row = (parts[0, 0] + parts[1, 0]).astype(jnp.int32)
    cnt_loop = (parts[0, 1] + parts[1, 1]).astype(jnp.int32)
    has_loop = cnt_loop > 0
    self_idx = jnp.arange(N, dtype=row0.dtype)
    row = jnp.concatenate([row0, self_idx])
    col = jnp.concatenate([col0, self_idx])
    valid = jnp.concatenate([jnp.ones((E,), bool), ~has_loop])
    ew = valid.astype(jnp.float32)
    deg = (cnt_row + 1 - has_loop.astype(jnp.int32)).astype(jnp.float32)
    dinv = jnp.where(deg > 0, deg ** -0.5, 0.0)
    h = x @ W_gcn
    upd_x = _gather_norm_scale(h, row, col, ew, dinv)
    x_pool = jax.ops.segment_sum(upd_x, row, num_segments=N) + b_gcn
    x_pool_j = _gather_rows(x_pool, col)
    X_q = jax.ops.segment_max(x_pool_j, row, num_segments=N)
    Mq_node = X_q @ W_q + b_q
    G = jnp.concatenate([Mq_node, x_pool], axis=0)
    i2 = jnp.stack([row, col + N], axis=1).reshape(2 * EP)
    cat = _concat_gather(G, i2).reshape(EP, 2 * D)
    score = cat @ W_att + b_att
    score = jax.nn.leaky_relu(score, negative_slope=NEG_SLOPE).reshape(-1)
    score = jnp.where(valid, score, -jnp.inf)
    smax = jax.ops.segment_max(score, row, num_segments=N)
    sexp = jnp.exp(score - _gather_scalar(smax, row))
    ssum = jax.ops.segment_sum(sexp, row, num_segments=N)
    score = sexp / _gather_scalar(ssum, row)
    upd_o = _gather_scale(x, col, score)
    out = jax.ops.segment_sum(upd_o, row, num_segments=N)
    ew_nl = jnp.where(row != col, ew, 0.0)
    h_le = out @ le_w
    deg_le = (cnt_row - cnt_loop).astype(jnp.float32)
    upd_a = (ew_nl * _gather_scalar(h_le.reshape(-1), col))[:, None]
    aggr = jax.ops.segment_sum(upd_a, row, num_segments=N)
    fit = deg_le[:, None] * (out @ le1_W + le1_b) + aggr + (out @ le2_W + le2_b)
    fitness = jax.nn.sigmoid(fit).reshape(-1)
    _, perm = jax.lax.top_k(fitness, K)
    x_new = out[perm] * fitness[perm][:, None]
    in_perm = jnp.zeros((N,), bool).at[perm].set(True)
    n_idx = jnp.zeros((N,), jnp.int32).at[perm].set(jnp.arange(K, dtype=jnp.int32))
    s_val = _gather_scalar(in_perm.astype(jnp.float32), row) * score
    A = jnp.zeros((N, N), jnp.float32).at[row, col].add(ew)
    S = jnp.zeros((N, K), jnp.float32).at[col, _gather_scalar(n_idx, row)].add(s_val)
    Ab = A.astype(jnp.bfloat16)
    Sb = S.astype(jnp.bfloat16)
    T = _matmul(Ab, Sb, bk=1024)
    Epool = _atb_with_unit_diag(Sb, T.astype(jnp.bfloat16), bk=1024)
    return x_new, Epool, perm


# speedup vs baseline: 1.0005x; 1.0005x over previous
import functools

import jax
import jax.numpy as jnp
from jax import lax
from jax.experimental import pallas as pl
from jax.experimental.pallas import tpu as pltpu
from jax.experimental.pallas import tpu_sc as plsc

N = 4096
D = 128
E = 65536
K = 2048
NEG_SLOPE = 0.2

EP = E + N
NW = 32
EPW = EP // NW
CH = 128
NCH = EPW // CH


def _sc_mesh():
    return plsc.VectorSubcoreMesh(core_axis_name="c", subcore_axis_name="s")


def _wid():
    return lax.axis_index("s") * 2 + lax.axis_index("c")


def _concat_gather(G, i2):
    @functools.partial(
        pl.kernel,
        out_type=jax.ShapeDtypeStruct((2 * EP, D), jnp.float32),
        mesh=_sc_mesh(),
        scratch_types=[
            pltpu.VMEM((2 * CH,), jnp.int32),
            pltpu.VMEM((2 * CH, D), jnp.float32),
            pltpu.SemaphoreType.DMA,
        ],
    )
    def k(G_hbm, i2_hbm, out_hbm, i2b, gb, sem):
        base_w = _wid() * EPW

        def body(ch, c):
            eb = base_w + ch * CH
            pltpu.sync_copy(i2_hbm.at[pl.ds(2 * eb, 2 * CH)], i2b)
            pltpu.async_copy(G_hbm.at[i2b], gb, sem).wait()
            pltpu.sync_copy(gb, out_hbm.at[pl.ds(2 * eb, 2 * CH)])
            return c

        lax.fori_loop(0, NCH, body, 0)

    return k(G, i2)


def _edge_counts(row0, col0):
    EW2 = E // NW
    NC2 = EW2 // CH

    @functools.partial(
        pl.kernel,
        out_type=jax.ShapeDtypeStruct((2, 2, N), jnp.float32),
        mesh=_sc_mesh(),
        scratch_types=[
            pltpu.VMEM((CH,), jnp.int32),
            pltpu.VMEM((CH,), jnp.int32),
            pltpu.VMEM((CH,), jnp.float32),
            pltpu.VMEM((CH,), jnp.float32),
            pltpu.VMEM_SHARED((N,), jnp.float32),
            pltpu.VMEM_SHARED((N,), jnp.float32),
            pltpu.VMEM((N // 16,), jnp.float32),
            pltpu.SemaphoreType.DMA,
            pltpu.SemaphoreType.DMA,
        ],
    )
    def k(row_hbm, col_hbm, out_hbm, rowb, colb, onesb, loopb, sh_r, sh_l, zb, sem, sem2):
        cid = lax.axis_index("c")
        sid = lax.axis_index("s")
        wid = sid * 2 + cid
        for gg in range(N // 16 // 16):
            zb[pl.ds(gg * 16, 16)] = jnp.zeros((16,), jnp.float32)
        for gg in range(CH // 16):
            onesb[pl.ds(gg * 16, 16)] = jnp.ones((16,), jnp.float32)
        pltpu.sync_copy(zb, sh_r.at[pl.ds(sid * (N // 16), N // 16)])
        pltpu.sync_copy(zb, sh_l.at[pl.ds(sid * (N // 16), N // 16)])
        plsc.subcore_barrier()

        def body(ch, c):
            eb = wid * EW2 + ch * CH
            pltpu.sync_copy(row_hbm.at[pl.ds(eb, CH)], rowb)
            pltpu.sync_copy(col_hbm.at[pl.ds(eb, CH)], colb)
            for g in range(CH // 16):
                sl = pl.ds(g * 16, 16)
                loopb[sl] = jnp.where(rowb[sl] == colb[sl], 1.0, 0.0)
            pltpu.async_copy(onesb, sh_r.at[rowb], sem, add=True)
            pltpu.async_copy(loopb, sh_l.at[rowb], sem2, add=True)
            pltpu.make_async_copy(onesb, sh_r.at[rowb], sem).wait()
            pltpu.make_async_copy(loopb, sh_l.at[rowb], sem2).wait()
            return c

        lax.fori_loop(0, NC2, body, 0)
        plsc.subcore_barrier()

        @pl.when(sid == 0)
        def _():
            pltpu.sync_copy(sh_r, out_hbm.at[cid, 0])
            pltpu.sync_copy(sh_l, out_hbm.at[cid, 1])

    return k(row0, col0)


def _gather_scalar(src, idx):
    @functools.partial(
        pl.kernel,
        out_type=jax.ShapeDtypeStruct((EP,), src.dtype),
        mesh=_sc_mesh(),
        scratch_types=[
            pltpu.VMEM((CH,), jnp.int32),
            pltpu.VMEM((CH,), src.dtype),
            pltpu.SemaphoreType.DMA,
        ],
    )
    def k(src_hbm, idx_hbm, out_hbm, idxb, gb, sem):
        base_w = _wid() * EPW

        def body(ch, c):
            eb = base_w + ch * CH
            pltpu.sync_copy(idx_hbm.at[pl.ds(eb, CH)], idxb)
            pltpu.async_copy(src_hbm.at[idxb], gb, sem).wait()
            pltpu.sync_copy(gb, out_hbm.at[pl.ds(eb, CH)])
            return c

        lax.fori_loop(0, NCH, body, 0)

    return k(src, idx)


def _gather_rows(src, idx):
    @functools.partial(
        pl.kernel,
        out_type=jax.ShapeDtypeStruct((EP, D), jnp.float32),
        mesh=_sc_mesh(),
        scratch_types=[
            pltpu.VMEM((CH,), jnp.int32),
            pltpu.VMEM((CH, D), jnp.float32),
            pltpu.SemaphoreType.DMA,
        ],
    )
    def k(src_hbm, idx_hbm, out_hbm, idxb, gb, sem):
        base_w = _wid() * EPW

        def body(ch, c):
            eb = base_w + ch * CH
            pltpu.sync_copy(idx_hbm.at[pl.ds(eb, CH)], idxb)
            pltpu.async_copy(src_hbm.at[idxb], gb, sem).wait()
            pltpu.sync_copy(gb, out_hbm.at[pl.ds(eb, CH)])
            return c

        lax.fori_loop(0, NCH, body, 0)

    return k(src, idx)


def _gather_norm_scale(h, row, col, ew, dinv):
    @functools.partial(
        pl.kernel,
        out_type=jax.ShapeDtypeStruct((EP, D), jnp.float32),
        mesh=_sc_mesh(),
        scratch_types=[
            pltpu.VMEM((CH,), jnp.int32),
            pltpu.VMEM((CH,), jnp.int32),
            pltpu.VMEM((CH,), jnp.float32),
            pltpu.VMEM((CH,), jnp.float32),
            pltpu.VMEM((CH,), jnp.float32),
            pltpu.VMEM((CH, D), jnp.float32),
            pltpu.SemaphoreType.DMA,
            pltpu.SemaphoreType.DMA,
            pltpu.SemaphoreType.DMA,
        ],
    )
    def k(h_hbm, row_hbm, col_hbm, ew_hbm, dinv_hbm, out_hbm,
          rowb, colb, ewb, drb, dcb, gb, sem, sem2, sem3):
        base_w = _wid() * EPW

        def body(ch, c):
            eb = base_w + ch * CH
            pltpu.sync_copy(row_hbm.at[pl.ds(eb, CH)], rowb)
            pltpu.sync_copy(col_hbm.at[pl.ds(eb, CH)], colb)
            pltpu.sync_copy(ew_hbm.at[pl.ds(eb, CH)], ewb)
            pltpu.async_copy(dinv_hbm.at[rowb], drb, sem2)
            pltpu.async_copy(dinv_hbm.at[colb], dcb, sem3)
            pltpu.async_copy(h_hbm.at[colb], gb, sem)
            pltpu.make_async_copy(dinv_hbm.at[rowb], drb, sem2).wait()
            pltpu.make_async_copy(dinv_hbm.at[colb], dcb, sem3).wait()
            for g in range(CH // 16):
                sl = pl.ds(g * 16, 16)
                ewb[sl] = (drb[sl] * ewb[sl]) * dcb[sl]
            pltpu.make_async_copy(h_hbm.at[colb], gb, sem).wait()
            for g in range(CH // 16):
                s16 = ewb[pl.ds(g * 16, 16)]
                for jj in range(16):
                    j = g * 16 + jj
                    sv = jnp.broadcast_to(s16[jj], (16,))
                    for dd in range(D // 16):
                        gb[j, pl.ds(dd * 16, 16)] = gb[j, pl.ds(dd * 16, 16)] * sv
            pltpu.sync_copy(gb, out_hbm.at[pl.ds(eb, CH)])
            return c

        lax.fori_loop(0, NCH, body, 0)

    return k(h, row, col, ew, dinv)


def _gather_scale(src, idx, scale):
    @functools.partial(
        pl.kernel,
        out_type=jax.ShapeDtypeStruct((EP, D), jnp.float32),
        mesh=_sc_mesh(),
        scratch_types=[
            pltpu.VMEM((CH,), jnp.int32),
            pltpu.VMEM((CH,), jnp.float32),
            pltpu.VMEM((CH, D), jnp.float32),
            pltpu.SemaphoreType.DMA,
        ],
    )
    def k(src_hbm, idx_hbm, sc_hbm, out_hbm, idxb, scb, gb, sem):
        base_w = _wid() * EPW

        def body(ch, c):
            eb = base_w + ch * CH
            pltpu.sync_copy(idx_hbm.at[pl.ds(eb, CH)], idxb)
            pltpu.sync_copy(sc_hbm.at[pl.ds(eb, CH)], scb)
            pltpu.async_copy(src_hbm.at[idxb], gb, sem).wait()
            for g in range(CH // 16):
                s16 = scb[pl.ds(g * 16, 16)]
                for jj in range(16):
                    j = g * 16 + jj
                    sv = jnp.broadcast_to(s16[jj], (16,))
                    for dd in range(D // 16):
                        gb[j, pl.ds(dd * 16, 16)] = gb[j, pl.ds(dd * 16, 16)] * sv
            pltpu.sync_copy(gb, out_hbm.at[pl.ds(eb, CH)])
            return c

        lax.fori_loop(0, NCH, body, 0)

    return k(src, idx, scale)


def _matmul_kernel(a_ref, b_ref, o_ref, acc_ref, *, n_k):
    k = pl.program_id(2)

    @pl.when(k == 0)
    def _():
        acc_ref[...] = jnp.zeros_like(acc_ref)

    acc_ref[...] += jnp.dot(a_ref[...], b_ref[...],
                            preferred_element_type=jnp.float32)

    @pl.when(k == n_k - 1)
    def _():
        o_ref[...] = acc_ref[...]


def _matmul(a, b, bm=512, bn=512, bk=512):
    m, k = a.shape
    k2, n = b.shape
    assert k == k2
    grid = (m // bm, n // bn, k // bk)
    return pl.pallas_call(
        functools.partial(_matmul_kernel, n_k=grid[2]),
        grid=grid,
        in_specs=[
            pl.BlockSpec((bm, bk), lambda i, j, kk: (i, kk)),
            pl.BlockSpec((bk, bn), lambda i, j, kk: (kk, j)),
        ],
        out_specs=pl.BlockSpec((bm, bn), lambda i, j, kk: (i, j)),
        out_shape=jax.ShapeDtypeStruct((m, n), jnp.float32),
        scratch_shapes=[pltpu.VMEM((bm, bn), jnp.float32)],
        compiler_params=pltpu.CompilerParams(
            dimension_semantics=("parallel", "parallel", "arbitrary")),
    )(a, b)


def _atb_diag_kernel(a_ref, b_ref, o_ref, acc_ref, *, n_k, bm, bn):
    i = pl.program_id(0)
    j = pl.program_id(1)
    k = pl.program_id(2)

    @pl.when(k == 0)
    def _():
        acc_ref[...] = jnp.zeros_like(acc_ref)

    acc_ref[...] += lax.dot_general(
        a_ref[...], b_ref[...], (((0,), (0,)), ((), ())),
        preferred_element_type=jnp.float32)

    @pl.when(k == n_k - 1)
    def _():
        rows = i * bm + lax.broadcasted_iota(jnp.int32, (bm, bn), 0)
        cols = j * bn + lax.broadcasted_iota(jnp.int32, (bm, bn), 1)
        o_ref[...] = jnp.where(rows == cols, 1.0, acc_ref[...])


def _atb_with_unit_diag(a, b, bm=512, bn=512, bk=512):
    k, m = a.shape
    k2, n = b.shape
    assert k == k2
    grid = (m // bm, n // bn, k // bk)
    return pl.pallas_call(
        functools.partial(_atb_diag_kernel, n_k=grid[2], bm=bm, bn=bn),
        grid=grid,
        in_specs=[
            pl.BlockSpec((bk, bm), lambda i, j, kk: (kk, i)),
            pl.BlockSpec((bk, bn), lambda i, j, kk: (kk, j)),
        ],
        out_specs=pl.BlockSpec((bm, bn), lambda i, j, kk: (i, j)),
        out_shape=jax.ShapeDtypeStruct((m, n), jnp.float32),
        scratch_shapes=[pltpu.VMEM((bm, bn), jnp.float32)],
        compiler_params=pltpu.CompilerParams(
            dimension_semantics=("parallel", "parallel", "arbitrary")),
    )(a, b)


def kernel(x, edge_index, W_gcn, b_gcn, W_q, b_q, W_att, b_att,
           le_w, le1_W, le1_b, le2_W, le2_b):
    row0, col0 = edge_index[0], edge_index[1]
    parts = _edge_counts(row0, col0)
    cnt_row = (parts[0, 0] + parts[1, 0]).astype(jnp.int32)
    cnt_loop = (parts[0, 1] + parts[1, 1]).astype(jnp.int32)
    has_loop = cnt_loop > 0
    self_idx = jnp.arange(N, dtype=row0.dtype)
    row = jnp.concatenate([row0, self_idx])
    col = jnp.concatenate([col0, self_idx])
    valid = jnp.concatenate([jnp.ones((E,), bool), ~has_loop])
    ew = valid.astype(jnp.float32)
    deg = (cnt_row + 1 - has_loop.astype(jnp.int32)).astype(jnp.float32)
    dinv = jnp.where(deg > 0, deg ** -0.5, 0.0)
    h = x @ W_gcn
    upd_x = _gather_norm_scale(h, row, col, ew, dinv)
    x_pool = jax.ops.segment_sum(upd_x, row, num_segments=N) + b_gcn
    x_pool_j = _gather_rows(x_pool, col)
    X_q = jax.ops.segment_max(x_pool_j, row, num_segments=N)
    Mq_node = X_q @ W_q + b_q
    G = jnp.concatenate([Mq_node, x_pool], axis=0)
    i2 = jnp.stack([row, col + N], axis=1).reshape(2 * EP)
    cat = _concat_gather(G, i2).reshape(EP, 2 * D)
    score = cat @ W_att + b_att
    score = jax.nn.leaky_relu(score, negative_slope=NEG_SLOPE).reshape(-1)
    score = jnp.where(valid, score, -jnp.inf)
    smax = jax.ops.segment_max(score, row, num_segments=N)
    sexp = jnp.exp(score - _gather_scalar(smax, row))
    ssum = jax.ops.segment_sum(sexp, row, num_segments=N)
    score = sexp / _gather_scalar(ssum, row)
    upd_o = _gather_scale(x, col, score)
    out = jax.ops.segment_sum(upd_o, row, num_segments=N)
    ew_nl = jnp.where(row != col, ew, 0.0)
    h_le = out @ le_w
    deg_le = (cnt_row - cnt_loop).astype(jnp.float32)
    upd_a = (ew_nl * _gather_scalar(h_le.reshape(-1), col))[:, None]
    aggr = jax.ops.segment_sum(upd_a, row, num_segments=N)
    fit = deg_le[:, None] * (out @ le1_W + le1_b) + aggr + (out @ le2_W + le2_b)
    fitness = jax.nn.sigmoid(fit).reshape(-1)
    _, perm = jax.lax.top_k(fitness, K)
    x_new = out[perm] * fitness[perm][:, None]
    in_perm = jnp.zeros((N,), bool).at[perm].set(True)
    n_idx = jnp.zeros((N,), jnp.int32).at[perm].set(jnp.arange(K, dtype=jnp.int32))
    s_val = _gather_scalar(in_perm.astype(jnp.float32), row) * score
    A = jnp.zeros((N, N), jnp.float32).at[row, col].add(ew)
    S = jnp.zeros((N, K), jnp.float32).at[col, _gather_scalar(n_idx, row)].add(s_val)
    Ab = A.astype(jnp.bfloat16)
    Sb = S.astype(jnp.bfloat16)
    T = _matmul(Ab, Sb, bk=1024)
    Epool = _atb_with_unit_diag(Sb, T.astype(jnp.bfloat16), bk=1024)
    return x_new, Epool, perm
